# baseline (speedup 1.0000x reference)
import jax
import jax.numpy as jnp
from jax import lax
from jax.experimental import pallas as pl
from jax.experimental.pallas import tpu as pltpu

_BN = 1024
_MERGE = 4
_NIMG = 8


def _body(nc_ref, grid_ref, maskfull_ref, mask_ref, feat_ref, table_ref,
          out_ref, bounds_ref):
    bi = pl.program_id(0)
    j = pl.program_id(1)
    nb_rows = maskfull_ref.shape[0]
    sub, lanes = maskfull_ref.shape[1], maskfull_ref.shape[2]

    @pl.when((bi == 0) & (j == 0))
    def _():
        l0 = lax.broadcasted_iota(jnp.int32, (lanes, lanes), 0)
        l1 = lax.broadcasted_iota(jnp.int32, (lanes, lanes), 1)
        tu = (l0 <= l1).astype(jnp.bfloat16)
        r0 = lax.broadcasted_iota(jnp.int32, (sub, sub), 0)
        r1 = lax.broadcasted_iota(jnp.int32, (sub, sub), 1)
        tls = (r0 > r1).astype(jnp.bfloat16)
        for b_ in range(nb_rows):
            m8 = maskfull_ref[b_].astype(jnp.bfloat16)
            pref = lax.dot_general(
                m8, tu, (((1,), (0,)), ((), ())),
                preferred_element_type=jnp.float32)
            rows = pref[:, lanes - 1:lanes].astype(jnp.bfloat16)
            off = lax.dot_general(
                tls, rows, (((1,), (0,)), ((), ())),
                preferred_element_type=jnp.float32)
            cnt = pref + off
            c = jnp.int32(0)
            for i in range(_NIMG):
                nt = (grid_ref[i, 0] * grid_ref[i, 1] * grid_ref[i, 2]) // _MERGE
                c = c + nt
                bj = jnp.sum((cnt <= c.astype(jnp.float32)).astype(jnp.int32))
                bounds_ref[b_ * _NIMG + i] = bj

    m = mask_ref[0, 0]
    bn = m.shape[0]
    pos = j * bn + lax.broadcasted_iota(jnp.int32, (bn, 1), 0)

    nc = nc_ref[0]
    ncs = jnp.maximum(nc, 1)
    k_iota = lax.broadcasted_iota(jnp.int32, (1, _NIMG), 1)

    prev = (m > 0) & (nc > 1)
    onehot = jnp.zeros((bn, _NIMG), jnp.bfloat16)
    for i in range(_NIMG):
        bj = bounds_ref[bi * _NIMG + i]
        lt = pos < bj
        ind = prev & lt
        sel = k_iota == (jnp.int32(i) % ncs)
        onehot = onehot + (ind & sel).astype(jnp.bfloat16)
        prev = prev & jnp.logical_not(lt)

    emb = lax.dot_general(
        onehot, table_ref[...], (((1,), (0,)), ((), ())),
        preferred_element_type=jnp.float32,
    )
    out_ref[0] = feat_ref[0] + emb


def _run(features, mask_i32, grid_i32, nc_arr, table_bf16):
    b, n, d = features.shape
    nb = n // _BN
    mask4 = mask_i32.reshape(b, nb, _BN, 1)
    mask_full = mask_i32.reshape(b, n // 128, 128)
    return pl.pallas_call(
        _body,
        grid=(b, nb),
        in_specs=[
            pl.BlockSpec(memory_space=pltpu.SMEM),
            pl.BlockSpec(memory_space=pltpu.SMEM),
            pl.BlockSpec((b, n // 128, 128), lambda b_, j: (0, 0, 0)),
            pl.BlockSpec((1, 1, _BN, 1), lambda b_, j: (b_, j, 0, 0)),
            pl.BlockSpec((1, _BN, d), lambda b_, j: (b_, j, 0)),
            pl.BlockSpec((_NIMG, d), lambda b_, j: (0, 0)),
        ],
        out_specs=pl.BlockSpec((1, _BN, d), lambda b_, j: (b_, j, 0)),
        out_shape=jax.ShapeDtypeStruct((b, n, d), features.dtype),
        scratch_shapes=[pltpu.SMEM((b * _NIMG,), jnp.int32)],
    )(nc_arr, grid_i32, mask_full, mask4, features, table_bf16)


def kernel(features, vision_mask, image_grid_thw, num_cameras, camera_table):
    nc_arr = jnp.asarray(num_cameras, jnp.int32).reshape(1)
    grid_i32 = jnp.asarray(image_grid_thw, jnp.int32)
    mask_i32 = jnp.asarray(vision_mask, jnp.int32)
    table_bf16 = camera_table.astype(jnp.bfloat16)
    return _run(features, mask_i32, grid_i32, nc_arr, table_bf16)
